# Initial kernel scaffold; baseline (speedup 1.0000x reference)
import jax
import jax.numpy as jnp
from jax.experimental import pallas as pl


def kernel(boxes, scores):
    raise NotImplementedError("write your pallas kernel here")


# speedup vs baseline: 679.4353x; 679.4353x over previous
import jax
import jax.numpy as jnp
from jax.experimental import pallas as pl
from jax.experimental.pallas import tpu as pltpu

_PRE = 6000
_POST = 300
_TH = 0.7
_T = 512
_P = 6144
_NT = _P // _T

_DN = (((1,), (0,)), ((), ()))


def _sup_mask(sx1, sy1, sx2, sy2, tx1, ty1, tx2, ty2):
    xx1 = jnp.maximum(sx1, tx1)
    yy1 = jnp.maximum(sy1, ty1)
    xx2 = jnp.minimum(sx2, tx2)
    yy2 = jnp.minimum(sy2, ty2)
    inter = jnp.clip(xx2 - xx1, 0.0) * jnp.clip(yy2 - yy1, 0.0)
    sa = (sx2 - sx1) * (sy2 - sy1)
    ta = (tx2 - tx1) * (ty2 - ty1)
    iou = inter / (sa + ta - inter + 1e-9)
    return jnp.where(iou > _TH, 1.0, 0.0).astype(jnp.float32)


def _nms_body(bx, bxt, keep_ref, cnt):
    keep_ref[...] = jnp.zeros((1, _P), jnp.float32)
    cnt[0] = 0

    def tile_coords_rows(t0):
        return (bx[pl.ds(t0, _T), 0:1], bx[pl.ds(t0, _T), 1:2],
                bx[pl.ds(t0, _T), 2:3], bx[pl.ds(t0, _T), 3:4])

    def tile_coords_cols(t0):
        return (bxt[0:1, pl.ds(t0, _T)], bxt[1:2, pl.ds(t0, _T)],
                bxt[2:3, pl.ds(t0, _T)], bxt[3:4, pl.ds(t0, _T)])

    def process_tile(r):
        t0 = r * _T
        tc = tile_coords_cols(t0)
        deadsum = jnp.zeros((1, _T), jnp.float32)
        for c in range(r):
            c0 = c * _T
            sup = _sup_mask(*tile_coords_rows(c0), *tc)
            kc = keep_ref[0:1, pl.ds(c0, _T)]
            deadsum = deadsum + jax.lax.dot_general(
                kc, sup, _DN, preferred_element_type=jnp.float32)
        supm = _sup_mask(*tile_coords_rows(t0), *tc)
        ii = jax.lax.broadcasted_iota(jnp.int32, (_T, _T), 0)
        jj = jax.lax.broadcasted_iota(jnp.int32, (_T, _T), 1)
        m = jnp.where(ii < jj, supm, 0.0)

        dead0 = jnp.where(deadsum > 0.0, 1.0, 0.0)
        kept0 = jnp.zeros((1, _T), jnp.float32)
        und0 = 1.0 - dead0

        def cond(s):
            _, _, u = s
            return jnp.sum(u) > 0.0

        def body(s):
            k, d, u = s
            alive = k + u
            a = jax.lax.dot_general(alive, m, _DN,
                                    preferred_element_type=jnp.float32)
            b = jax.lax.dot_general(k, m, _DN,
                                    preferred_element_type=jnp.float32)
            newk = u * jnp.where(a == 0.0, 1.0, 0.0)
            newd = u * jnp.where(b > 0.0, 1.0, 0.0)
            return (k + newk, d + newd, u - newk - newd)

        k, _, _ = jax.lax.while_loop(cond, body, (kept0, dead0, und0))
        keep_ref[0:1, pl.ds(t0, _T)] = k
        lane = jax.lax.broadcasted_iota(jnp.int32, (1, _T), 1)
        realm = jnp.where(lane + t0 < _PRE, 1.0, 0.0)
        cnt[0] = cnt[0] + jnp.sum(k * realm).astype(jnp.int32)

    for r in range(_NT):
        if r == 0:
            process_tile(0)
        else:
            pl.when(cnt[0] < _POST)(lambda r=r: process_tile(r))


def _nms_keep_pallas(bx, bxt):
    return pl.pallas_call(
        _nms_body,
        out_shape=jax.ShapeDtypeStruct((1, _P), jnp.float32),
        scratch_shapes=[pltpu.SMEM((1,), jnp.int32)],
    )(bx, bxt)


def kernel(boxes, scores):
    obj = jax.nn.softmax(scores, axis=1)[:, 1]
    top_scores, idx = jax.lax.top_k(obj, _PRE)
    top_boxes = jnp.take(boxes, idx, axis=0)
    pad = jnp.tile(jnp.array([[0.0, 0.0, 1.0, 1.0]], jnp.float32),
                   (_P - _PRE, 1))
    bx = jnp.concatenate([top_boxes, pad], axis=0)
    keep_row = _nms_keep_pallas(bx, bx.T)
    keep = keep_row[0, :_PRE] > 0.5
    masked = jnp.where(keep, top_scores, -1e9)
    sel_scores, sel_idx = jax.lax.top_k(masked, _POST)
    sel_boxes = jnp.take(top_boxes, sel_idx, axis=0)
    return jnp.concatenate([sel_boxes, sel_scores[:, None]], axis=1)


# speedup vs baseline: 1291.8969x; 1.9014x over previous
import jax
import jax.numpy as jnp
from jax.experimental import pallas as pl
from jax.experimental.pallas import tpu as pltpu

_PRE = 6000
_POST = 300
_TH = 0.7
_T = 512
_P = 6144
_NT = _P // _T

_DN = (((1,), (0,)), ((), ()))


def _sup_mask(sx1, sy1, sx2, sy2, tx1, ty1, tx2, ty2):
    xx1 = jnp.maximum(sx1, tx1)
    yy1 = jnp.maximum(sy1, ty1)
    xx2 = jnp.minimum(sx2, tx2)
    yy2 = jnp.minimum(sy2, ty2)
    inter = jnp.clip(xx2 - xx1, 0.0) * jnp.clip(yy2 - yy1, 0.0)
    sa = (sx2 - sx1) * (sy2 - sy1)
    ta = (tx2 - tx1) * (ty2 - ty1)
    iou = inter / (sa + ta - inter + 1e-9)
    return jnp.where(iou > _TH, 1.0, 0.0).astype(jnp.float32)


def _greedy_fixpoint(m, dead0):
    kept0 = jnp.zeros_like(dead0)
    und0 = 1.0 - dead0

    def cond(s):
        _, _, u = s
        return jnp.sum(u) > 0.0

    def round_(s):
        k, d, u = s
        ab = jax.lax.dot_general(jnp.concatenate([k + u, k], axis=0), m,
                                 _DN, preferred_element_type=jnp.float32)
        a, b = ab[0:1, :], ab[1:2, :]
        newk = u * jnp.where(a == 0.0, 1.0, 0.0)
        newd = u * jnp.where(b > 0.0, 1.0, 0.0)
        return (k + newk, d + newd, u - newk - newd)

    def body(s):
        return round_(round_(s))

    k, _, _ = jax.lax.while_loop(cond, body, (kept0, dead0, und0))
    return k


def _upper_iota(t):
    ii = jax.lax.broadcasted_iota(jnp.int32, (t, t), 0)
    jj = jax.lax.broadcasted_iota(jnp.int32, (t, t), 1)
    return ii < jj


def _nms_small_body(bx, bxt, keep_ref):
    sc = (bx[:, 0:1], bx[:, 1:2], bx[:, 2:3], bx[:, 3:4])
    tc = (bxt[0:1, :], bxt[1:2, :], bxt[2:3, :], bxt[3:4, :])
    supm = _sup_mask(*sc, *tc)
    m = jnp.where(_upper_iota(_T), supm, 0.0)
    keep_ref[...] = _greedy_fixpoint(m, jnp.zeros((1, _T), jnp.float32))


def _nms_full_body(bx, bxt, keep_ref, cnt):
    keep_ref[...] = jnp.zeros((1, _P), jnp.float32)
    cnt[0] = 0

    def tile_coords_rows(t0):
        return (bx[pl.ds(t0, _T), 0:1], bx[pl.ds(t0, _T), 1:2],
                bx[pl.ds(t0, _T), 2:3], bx[pl.ds(t0, _T), 3:4])

    def tile_coords_cols(t0):
        return (bxt[0:1, pl.ds(t0, _T)], bxt[1:2, pl.ds(t0, _T)],
                bxt[2:3, pl.ds(t0, _T)], bxt[3:4, pl.ds(t0, _T)])

    def process_tile(r):
        t0 = r * _T
        tc = tile_coords_cols(t0)
        deadsum = jnp.zeros((1, _T), jnp.float32)
        for c in range(r):
            c0 = c * _T
            sup = _sup_mask(*tile_coords_rows(c0), *tc)
            kc = keep_ref[0:1, pl.ds(c0, _T)]
            deadsum = deadsum + jax.lax.dot_general(
                kc, sup, _DN, preferred_element_type=jnp.float32)
        supm = _sup_mask(*tile_coords_rows(t0), *tc)
        m = jnp.where(_upper_iota(_T), supm, 0.0)
        dead0 = jnp.where(deadsum > 0.0, 1.0, 0.0)
        k = _greedy_fixpoint(m, dead0)
        keep_ref[0:1, pl.ds(t0, _T)] = k
        lane = jax.lax.broadcasted_iota(jnp.int32, (1, _T), 1)
        realm = jnp.where(lane + t0 < _PRE, 1.0, 0.0)
        cnt[0] = cnt[0] + jnp.sum(k * realm).astype(jnp.int32)

    for r in range(_NT):
        if r == 0:
            process_tile(0)
        else:
            pl.when(cnt[0] < _POST)(lambda r=r: process_tile(r))


def _nms_keep_small(bx, bxt):
    return pl.pallas_call(
        _nms_small_body,
        out_shape=jax.ShapeDtypeStruct((1, _T), jnp.float32),
    )(bx, bxt)


def _nms_keep_full(bx, bxt):
    return pl.pallas_call(
        _nms_full_body,
        out_shape=jax.ShapeDtypeStruct((1, _P), jnp.float32),
        scratch_shapes=[pltpu.SMEM((1,), jnp.int32)],
    )(bx, bxt)


def _assemble(top_boxes, top_scores, keep):
    masked = jnp.where(keep, top_scores, -1e9)
    sel_scores, sel_idx = jax.lax.top_k(masked, _POST)
    sel_boxes = jnp.take(top_boxes, sel_idx, axis=0)
    return jnp.concatenate([sel_boxes, sel_scores[:, None]], axis=1)


def kernel(boxes, scores):
    obj = jax.nn.softmax(scores, axis=1)[:, 1]

    ts512, idx512 = jax.lax.top_k(obj, _T)
    tb512 = jnp.take(boxes, idx512, axis=0)
    keep512_row = _nms_keep_small(tb512, tb512.T)
    keep512 = keep512_row[0, :] > 0.5
    kept_cnt = jnp.sum(keep512_row).astype(jnp.int32)

    def fast(_):
        return _assemble(tb512, ts512, keep512)

    def slow(_):
        top_scores, idx = jax.lax.top_k(obj, _PRE)
        top_boxes = jnp.take(boxes, idx, axis=0)
        pad = jnp.tile(jnp.array([[0.0, 0.0, 1.0, 1.0]], jnp.float32),
                       (_P - _PRE, 1))
        bx = jnp.concatenate([top_boxes, pad], axis=0)
        keep_row = _nms_keep_full(bx, bx.T)
        keep = keep_row[0, :_PRE] > 0.5
        return _assemble(top_boxes, top_scores, keep)

    return jax.lax.cond(kept_cnt >= _POST, fast, slow, None)
